# baseline (speedup 1.0000x reference)
import jax
import jax.numpy as jnp
from jax import lax
from jax.experimental import pallas as pl
from jax.experimental.pallas import tpu as pltpu
from jax.experimental.pallas import tpu_sc as plsc

NUM_STATES = 16
NUM_ACTIONS = 8
BATCH = 16384
OBS_DIM = 4

NC = 1
NS = 16
L = 16
NW = NC * NS
B_PER_W = BATCH // NW
CBLK = BATCH // 128
CB_PER_W = B_PER_W // 128
GROUPS = B_PER_W // L


def _body(obs_hbm, table_hbm, out_hbm, idx_v, tab_v, out_v, sem1, sem2):
    wid = lax.axis_index("s") * NC + lax.axis_index("c")

    c1 = pltpu.make_async_copy(
        obs_hbm.at[pl.ds(wid * CB_PER_W, CB_PER_W), pl.ds(0, 1)], idx_v, sem1
    )
    c2 = pltpu.make_async_copy(table_hbm, tab_v, sem2)
    c1.start()
    c2.start()
    c2.wait()

    cols = [tab_v[j, pl.ds(0, L)] for j in range(NUM_ACTIONS)]
    c1.wait()

    def group(g, _):
        cb = g // (128 // L)
        lo = (g % (128 // L)) * L
        states = idx_v[cb, 0, pl.ds(lo, L)]
        for j in range(NUM_ACTIONS):
            out_v[cb, j, pl.ds(lo, L)] = jnp.take_along_axis(
                cols[j], states, axis=0
            )
        return _

    lax.fori_loop(0, GROUPS, group, 0, unroll=2)

    pltpu.sync_copy(out_v, out_hbm.at[pl.ds(wid * CB_PER_W, CB_PER_W)])


_mesh = plsc.VectorSubcoreMesh(
    core_axis_name="c", subcore_axis_name="s", num_cores=NC, num_subcores=NS
)
_lookup = pl.kernel(
    _body,
    out_type=jax.ShapeDtypeStruct((CBLK, NUM_ACTIONS, 128), jnp.float32),
    mesh=_mesh,
    scratch_types=[
        pltpu.VMEM((CB_PER_W, 1, 128), jnp.int32),
        pltpu.VMEM((NUM_ACTIONS, 128), jnp.float32),
        pltpu.VMEM((CB_PER_W, NUM_ACTIONS, 128), jnp.float32),
        pltpu.SemaphoreType.DMA,
        pltpu.SemaphoreType.DMA,
    ],
    compiler_params=pltpu.CompilerParams(
        needs_layout_passes=False,
        use_tc_tiling_on_sc=False,
        disable_bounds_checks=True,
        disable_semaphore_checks=True,
    ),
)


@jax.jit
def kernel(obs, table):
    obs_view = obs.T.reshape(OBS_DIM, CBLK, 128).transpose(1, 0, 2)
    table_t = jnp.pad(table.T, ((0, 0), (0, 128 - NUM_STATES)))
    a = _lookup(obs_view, table_t)
    return a.transpose(1, 0, 2).reshape(NUM_ACTIONS, BATCH).T


# speedup vs baseline: 1.0083x; 1.0083x over previous
import jax
import jax.numpy as jnp
from jax import lax
from jax.experimental import pallas as pl
from jax.experimental.pallas import tpu as pltpu
from jax.experimental.pallas import tpu_sc as plsc

NUM_STATES = 16
NUM_ACTIONS = 8
BATCH = 16384
OBS_DIM = 4

NC = 1
NS = 16
L = 16
NW = NC * NS
B_PER_W = BATCH // NW
CBLK = BATCH // 128
CB_PER_W = B_PER_W // 128
GROUPS = B_PER_W // L


def _body(obs_hbm, table_hbm, out_hbm, idx_v, tab_v, out_v, sem1, sem2):
    wid = lax.axis_index("s") * NC + lax.axis_index("c")

    c1 = pltpu.make_async_copy(
        obs_hbm.at[pl.ds(wid * CB_PER_W, CB_PER_W), pl.ds(0, 1)], idx_v, sem1
    )
    c2 = pltpu.make_async_copy(table_hbm, tab_v, sem2)
    c1.start()
    c2.start()
    c2.wait()

    lanes = lax.iota(jnp.int32, L)
    cols = [plsc.load_gather(tab_v, [lanes * NUM_ACTIONS + j])
            for j in range(NUM_ACTIONS)]
    c1.wait()

    def group(g, _):
        cb = g // (128 // L)
        lo = (g % (128 // L)) * L
        states = idx_v[cb, 0, pl.ds(lo, L)]
        for j in range(NUM_ACTIONS):
            out_v[cb, j, pl.ds(lo, L)] = jnp.take_along_axis(
                cols[j], states, axis=0
            )
        return _

    lax.fori_loop(0, GROUPS, group, 0, unroll=2)

    pltpu.sync_copy(out_v, out_hbm.at[pl.ds(wid * CB_PER_W, CB_PER_W)])


_mesh = plsc.VectorSubcoreMesh(
    core_axis_name="c", subcore_axis_name="s", num_cores=NC, num_subcores=NS
)
_lookup = pl.kernel(
    _body,
    out_type=jax.ShapeDtypeStruct((CBLK, NUM_ACTIONS, 128), jnp.float32),
    mesh=_mesh,
    scratch_types=[
        pltpu.VMEM((CB_PER_W, 1, 128), jnp.int32),
        pltpu.VMEM((NUM_STATES * NUM_ACTIONS,), jnp.float32),
        pltpu.VMEM((CB_PER_W, NUM_ACTIONS, 128), jnp.float32),
        pltpu.SemaphoreType.DMA,
        pltpu.SemaphoreType.DMA,
    ],
    compiler_params=pltpu.CompilerParams(
        needs_layout_passes=False,
        use_tc_tiling_on_sc=False,
        disable_bounds_checks=True,
        disable_semaphore_checks=True,
    ),
)


@jax.jit
def kernel(obs, table):
    obs_view = obs.T.reshape(OBS_DIM, CBLK, 128).transpose(1, 0, 2)
    a = _lookup(obs_view, table.reshape(NUM_STATES * NUM_ACTIONS))
    return a.transpose(1, 0, 2).reshape(NUM_ACTIONS, BATCH).T


# speedup vs baseline: 1.0113x; 1.0030x over previous
import jax
import jax.numpy as jnp
from jax import lax
from jax.experimental import pallas as pl
from jax.experimental.pallas import tpu as pltpu
from jax.experimental.pallas import tpu_sc as plsc

NUM_STATES = 16
NUM_ACTIONS = 8
BATCH = 16384
OBS_DIM = 4

NC = 1
NS = 16
L = 16
NW = NC * NS
B_PER_W = BATCH // NW
CBLK = BATCH // 128
CB_PER_W = B_PER_W // 128
GROUPS = B_PER_W // L


def _body(obs_hbm, table_hbm, out_hbm, idx_v, tab_v, out_v, sem1, sem2):
    wid = lax.axis_index("s") * NC + lax.axis_index("c")

    c1 = pltpu.make_async_copy(
        obs_hbm.at[pl.ds(wid * CB_PER_W, CB_PER_W), pl.ds(0, 1)], idx_v, sem1
    )
    c2 = pltpu.make_async_copy(table_hbm, tab_v, sem2)
    c1.start()
    c2.start()
    c2.wait()

    lanes = lax.iota(jnp.int32, L)
    cols = [plsc.load_gather(tab_v, [lanes * NUM_ACTIONS + j])
            for j in range(NUM_ACTIONS)]
    c1.wait()

    def group(g, _):
        cb = g // (128 // L)
        lo = (g % (128 // L)) * L
        states = idx_v[cb, 0, pl.ds(lo, L)]
        for j in range(NUM_ACTIONS):
            out_v[cb, j, pl.ds(lo, L)] = jnp.take_along_axis(
                cols[j], states, axis=0
            )
        return _

    lax.fori_loop(0, GROUPS, group, 0, unroll=2)

    pltpu.sync_copy(out_v, out_hbm.at[pl.ds(wid * CB_PER_W, CB_PER_W)])


_mesh = plsc.VectorSubcoreMesh(
    core_axis_name="c", subcore_axis_name="s", num_cores=NC, num_subcores=NS
)
_lookup = pl.kernel(
    _body,
    out_type=jax.ShapeDtypeStruct((CBLK, NUM_ACTIONS, 128), jnp.float32),
    mesh=_mesh,
    scratch_types=[
        pltpu.VMEM((CB_PER_W, 1, 128), jnp.int32),
        pltpu.VMEM((NUM_STATES * NUM_ACTIONS,), jnp.float32),
        pltpu.VMEM((CB_PER_W, NUM_ACTIONS, 128), jnp.float32),
        pltpu.SemaphoreType.DMA,
        pltpu.SemaphoreType.DMA,
    ],
    compiler_params=pltpu.CompilerParams(
        needs_layout_passes=False,
        use_tc_tiling_on_sc=False,
        disable_bounds_checks=True,
        disable_semaphore_checks=True,
        skip_device_barrier=True,
    ),
)


@jax.jit
def kernel(obs, table):
    obs_view = obs.T.reshape(OBS_DIM, CBLK, 128).transpose(1, 0, 2)
    a = _lookup(obs_view, table.reshape(NUM_STATES * NUM_ACTIONS))
    return a.transpose(1, 0, 2).reshape(NUM_ACTIONS, BATCH).T


# speedup vs baseline: 1.0122x; 1.0009x over previous
import jax
import jax.numpy as jnp
from jax import lax
from jax.experimental import pallas as pl
from jax.experimental.pallas import tpu as pltpu
from jax.experimental.pallas import tpu_sc as plsc

NUM_STATES = 16
NUM_ACTIONS = 8
BATCH = 16384
OBS_DIM = 4

NC = 1
NS = 16
L = 16
NW = NC * NS
B_PER_W = BATCH // NW
CBLK = BATCH // 128
CB_PER_W = B_PER_W // 128
GROUPS = B_PER_W // L


def _body(obs_hbm, table_hbm, out_hbm, idx_v, tab_v, out_v, sem1, sem2):
    wid = lax.axis_index("s") * NC + lax.axis_index("c")

    c1 = pltpu.make_async_copy(
        obs_hbm.at[pl.ds(wid * CB_PER_W, CB_PER_W), pl.ds(0, 1)], idx_v, sem1
    )
    c2 = pltpu.make_async_copy(table_hbm, tab_v, sem2)
    c1.start()
    c2.start()
    c2.wait()

    lanes = lax.iota(jnp.int32, L)
    cols = [plsc.load_gather(tab_v, [lanes * NUM_ACTIONS + j])
            for j in range(NUM_ACTIONS)]
    c1.wait()

    def group(g, _):
        cb = g // (128 // L)
        lo = (g % (128 // L)) * L
        states = idx_v[cb, 0, pl.ds(lo, L)]
        for j in range(NUM_ACTIONS):
            out_v[cb, j, pl.ds(lo, L)] = jnp.take_along_axis(
                cols[j], states, axis=0
            )
        return _

    lax.fori_loop(0, GROUPS, group, 0, unroll=2)

    pltpu.sync_copy(out_v, out_hbm.at[pl.ds(wid * CB_PER_W, CB_PER_W)])


_mesh = plsc.VectorSubcoreMesh(
    core_axis_name="c", subcore_axis_name="s", num_cores=NC, num_subcores=NS
)
_lookup = pl.kernel(
    _body,
    out_type=jax.ShapeDtypeStruct((CBLK, NUM_ACTIONS, 128), jnp.float32),
    mesh=_mesh,
    scratch_types=[
        pltpu.VMEM((CB_PER_W, 1, 128), jnp.int32),
        pltpu.VMEM((NUM_STATES * NUM_ACTIONS,), jnp.float32),
        pltpu.VMEM((CB_PER_W, NUM_ACTIONS, 128), jnp.float32),
        pltpu.SemaphoreType.DMA,
        pltpu.SemaphoreType.DMA,
    ],
    compiler_params=pltpu.CompilerParams(
        needs_layout_passes=False,
        use_tc_tiling_on_sc=False,
    ),
)


@jax.jit
def kernel(obs, table):
    obs_view = obs.T.reshape(OBS_DIM, CBLK, 128).transpose(1, 0, 2)
    a = _lookup(obs_view, table.reshape(NUM_STATES * NUM_ACTIONS))
    return a.transpose(1, 0, 2).reshape(NUM_ACTIONS, BATCH).T
